# Initial kernel scaffold; baseline (speedup 1.0000x reference)
import functools

import jax
import jax.numpy as jnp
import numpy as np
from jax import lax
from jax.experimental import pallas as pl
from jax.experimental.pallas import tpu as pltpu
from jax.experimental.pallas import tpu_sc as plsc

B = 16384
D = 64
L = 20
NEG = 20
NW = 32
RW = B // NW
C = 8
NCH = RW // C
OUTW = 16 * (1 + NEG)


def _sc_partials(target_emb, context_emb, output_emb, gidx, ctidx, ctxidx, negidx):
  mesh = plsc.VectorSubcoreMesh(core_axis_name="c", subcore_axis_name="s")

  @functools.partial(
      pl.kernel,
      mesh=mesh,
      out_type=jax.ShapeDtypeStruct((B, OUTW), jnp.float32),
      scratch_types=[
          pltpu.VMEM((RW,), jnp.int32),
          pltpu.VMEM((RW,), jnp.int32),
          pltpu.VMEM((RW * L,), jnp.int32),
          pltpu.VMEM((RW * NEG,), jnp.int32),
          pltpu.VMEM((2, C, D), jnp.float32),
          pltpu.VMEM((2, C * L, D), jnp.float32),
          pltpu.VMEM((2, C, 2 * D), jnp.float32),
          pltpu.VMEM((2, C * NEG, 2 * D), jnp.float32),
          pltpu.VMEM((2, C, OUTW), jnp.float32),
          pltpu.SemaphoreType.DMA,
          pltpu.SemaphoreType.DMA,
          pltpu.SemaphoreType.DMA,
          pltpu.SemaphoreType.DMA,
      ],
  )
  def k(tgt_hbm, ctx_hbm, oemb_hbm, gidx_hbm, ctidx_hbm, ctxidx_hbm,
        negidx_hbm, out_hbm, gidx_v, ctidx_v, ctxidx_v, negidx_v,
        tgt_buf, ctx_buf, ct_buf, neg_buf, out_buf,
        sem_in0, sem_in1, sem_out0, sem_out1):
    wid = lax.axis_index("s") * 2 + lax.axis_index("c")
    base = wid * RW

    pltpu.sync_copy(gidx_hbm.at[pl.ds(base, RW)], gidx_v)
    pltpu.sync_copy(ctidx_hbm.at[pl.ds(base, RW)], ctidx_v)
    pltpu.sync_copy(ctxidx_hbm.at[pl.ds(base * L, RW * L)], ctxidx_v)
    pltpu.sync_copy(negidx_hbm.at[pl.ds(base * NEG, RW * NEG)], negidx_v)

    sems_in = (sem_in0, sem_in1)
    sems_out = (sem_out0, sem_out1)

    def chunk_copies(c, b):
      sem = sems_in[b]
      cps = [
          pltpu.make_async_copy(
              tgt_hbm.at[gidx_v.at[pl.ds(c * C, C)]], tgt_buf.at[b], sem),
          pltpu.make_async_copy(
              oemb_hbm.at[ctidx_v.at[pl.ds(c * C, C)]], ct_buf.at[b], sem),
      ]
      half = C * L // 2
      for h in range(2):
        cps.append(pltpu.make_async_copy(
            ctx_hbm.at[ctxidx_v.at[pl.ds(c * C * L + h * half, half)]],
            ctx_buf.at[b, pl.ds(h * half, half)], sem))
        cps.append(pltpu.make_async_copy(
            oemb_hbm.at[negidx_v.at[pl.ds(c * C * NEG + h * half, half)]],
            neg_buf.at[b, pl.ds(h * half, half)], sem))
      return cps

    def issue_chunk(c, b):
      for cp in chunk_copies(c, b):
        cp.start()

    def wait_chunk(c, b):
      for cp in chunk_copies(c, b):
        cp.wait()

    def out_copy(c, b):
      return pltpu.make_async_copy(
          out_buf.at[b], out_hbm.at[pl.ds(base + c * C, C)], sems_out[b])

    def compute_chunk(b):
      def row(r, carry):
        rl = r * L
        stack = [tgt_buf[b, r, pl.ds(16 * k, 16)] for k in range(4)]
        for k in range(4):
          acc = ctx_buf[b, rl, pl.ds(16 * k, 16)]
          for l in range(1, L):
            acc = acc + ctx_buf[b, rl + l, pl.ds(16 * k, 16)]
          stack.append(acc)
        p = stack[0] * ct_buf[b, r, pl.ds(0, 16)]
        for k in range(1, 8):
          p = p + stack[k] * ct_buf[b, r, pl.ds(16 * k, 16)]
        out_buf[b, r, pl.ds(0, 16)] = p
        for n in range(NEG):
          q = stack[0] * neg_buf[b, rl + n, pl.ds(0, 16)]
          for k in range(1, 8):
            q = q + stack[k] * neg_buf[b, rl + n, pl.ds(16 * k, 16)]
          out_buf[b, r, pl.ds(16 * (n + 1), 16)] = q
        return carry
      lax.fori_loop(0, C, row, 0)

    issue_chunk(0, 0)

    def body(i, carry):
      for b in (0, 1):
        c = 2 * i + b

        @pl.when(c + 1 < NCH)
        def _issue():
          issue_chunk(c + 1, 1 - b)

        wait_chunk(c, b)

        @pl.when(c >= 2)
        def _drain():
          out_copy(c - 2, b).wait()

        compute_chunk(b)
        out_copy(c, b).start()
      return carry

    lax.fori_loop(0, NCH // 2, body, 0)
    out_copy(NCH - 2, 0).wait()
    out_copy(NCH - 1, 1).wait()

  return k(target_emb, context_emb, output_emb, gidx, ctidx, ctxidx, negidx)


def _softplus(x):
  return jnp.maximum(x, 0.0) + jnp.log1p(jnp.exp(-jnp.abs(x)))


def _tc_loss(parts, gmat):
  nblk = 8
  rows = B // nblk

  def body(p_ref, g_ref, o_ref):
    i = pl.program_id(0)
    d = jnp.dot(p_ref[...], g_ref[...], preferred_element_type=jnp.float32)
    d = jnp.clip(d, -10.0, 10.0)
    part = (jnp.sum(_softplus(-d[:, 0:1])) +
            jnp.sum(_softplus(d[:, 1:1 + NEG])))

    @pl.when(i == 0)
    def _init():
      o_ref[0, 0] = 0.0

    o_ref[0, 0] += part

    @pl.when(i == nblk - 1)
    def _fin():
      o_ref[0, 0] = o_ref[0, 0] * (1.0 / B)

  out = pl.pallas_call(
      body,
      grid=(nblk,),
      in_specs=[
          pl.BlockSpec((rows, OUTW), lambda i: (i, 0)),
          pl.BlockSpec((OUTW, 1 + NEG), lambda i: (0, 0)),
      ],
      out_specs=pl.BlockSpec(memory_space=pltpu.SMEM),
      out_shape=jax.ShapeDtypeStruct((1, 1), jnp.float32),
  )(parts, gmat)
  return out[0, 0]


_GMAT = np.repeat(np.eye(1 + NEG, dtype=np.float32), 16, axis=0)


def kernel(target_emb, context_emb, output_emb, pos_graph_emb,
           pos_context_target, pos_contexts, pos_negatives):
  gidx = jnp.asarray(pos_graph_emb, jnp.int32)
  ctidx = jnp.asarray(pos_context_target, jnp.int32)
  ctxidx = jnp.asarray(pos_contexts, jnp.int32).reshape(-1)
  negidx = jnp.asarray(pos_negatives, jnp.int32).reshape(-1)
  parts = _sc_partials(target_emb, context_emb, output_emb,
                       gidx, ctidx, ctxidx, negidx)
  return _tc_loss(parts, jnp.asarray(_GMAT))


# speedup vs baseline: 7.7369x; 7.7369x over previous
import functools

import jax
import jax.numpy as jnp
import numpy as np
from jax import lax
from jax.experimental import pallas as pl
from jax.experimental.pallas import tpu as pltpu
from jax.experimental.pallas import tpu_sc as plsc

B = 16384
D = 64
L = 20
NEG = 20
NW = 32
RW = B // NW
C = 8
NCH = RW // C
OUTW = 16 * (1 + NEG)


def _sc_partials(target_emb, context_emb, output_emb, gidx, ctidx, ctxidx, negidx):
  mesh = plsc.VectorSubcoreMesh(core_axis_name="c", subcore_axis_name="s")

  @functools.partial(
      pl.kernel,
      mesh=mesh,
      compiler_params=pltpu.CompilerParams(use_tc_tiling_on_sc=False),
      out_type=jax.ShapeDtypeStruct((B, OUTW), jnp.float32),
      scratch_types=[
          pltpu.VMEM((RW,), jnp.int32),
          pltpu.VMEM((RW,), jnp.int32),
          pltpu.VMEM((RW * L,), jnp.int32),
          pltpu.VMEM((RW * NEG,), jnp.int32),
          pltpu.VMEM((2, C, D), jnp.float32),
          pltpu.VMEM((2, C * L, D), jnp.float32),
          pltpu.VMEM((2, C, 2 * D), jnp.float32),
          pltpu.VMEM((2, C * NEG, 2 * D), jnp.float32),
          pltpu.VMEM((2, C, OUTW), jnp.float32),
          pltpu.SemaphoreType.DMA,
          pltpu.SemaphoreType.DMA,
          pltpu.SemaphoreType.DMA,
          pltpu.SemaphoreType.DMA,
      ],
  )
  def k(tgt_hbm, ctx_hbm, oemb_hbm, gidx_hbm, ctidx_hbm, ctxidx_hbm,
        negidx_hbm, out_hbm, gidx_v, ctidx_v, ctxidx_v, negidx_v,
        tgt_buf, ctx_buf, ct_buf, neg_buf, out_buf,
        sem_in0, sem_in1, sem_out0, sem_out1):
    wid = lax.axis_index("s") * 2 + lax.axis_index("c")
    base = wid * RW

    pltpu.sync_copy(gidx_hbm.at[pl.ds(base, RW)], gidx_v)
    pltpu.sync_copy(ctidx_hbm.at[pl.ds(base, RW)], ctidx_v)
    pltpu.sync_copy(ctxidx_hbm.at[pl.ds(base * L, RW * L)], ctxidx_v)
    pltpu.sync_copy(negidx_hbm.at[pl.ds(base * NEG, RW * NEG)], negidx_v)

    sems_in = (sem_in0, sem_in1)
    sems_out = (sem_out0, sem_out1)

    def chunk_copies(c, b):
      sem = sems_in[b]
      cps = [
          pltpu.make_async_copy(
              tgt_hbm.at[gidx_v.at[pl.ds(c * C, C)]], tgt_buf.at[b], sem),
          pltpu.make_async_copy(
              oemb_hbm.at[ctidx_v.at[pl.ds(c * C, C)]], ct_buf.at[b], sem),
      ]
      half = C * L // 2
      for h in range(2):
        cps.append(pltpu.make_async_copy(
            ctx_hbm.at[ctxidx_v.at[pl.ds(c * C * L + h * half, half)]],
            ctx_buf.at[b, pl.ds(h * half, half)], sem))
        cps.append(pltpu.make_async_copy(
            oemb_hbm.at[negidx_v.at[pl.ds(c * C * NEG + h * half, half)]],
            neg_buf.at[b, pl.ds(h * half, half)], sem))
      return cps

    def issue_chunk(c, b):
      for cp in chunk_copies(c, b):
        cp.start()

    def wait_chunk(c, b):
      for cp in chunk_copies(c, b):
        cp.wait()

    def out_copy(c, b):
      return pltpu.make_async_copy(
          out_buf.at[b], out_hbm.at[pl.ds(base + c * C, C)], sems_out[b])

    def compute_chunk(b):
      def row(r, carry):
        rl = r * L
        stack = [tgt_buf[b, r, pl.ds(16 * k, 16)] for k in range(4)]
        for k in range(4):
          acc = ctx_buf[b, rl, pl.ds(16 * k, 16)]
          for l in range(1, L):
            acc = acc + ctx_buf[b, rl + l, pl.ds(16 * k, 16)]
          stack.append(acc)
        p = stack[0] * ct_buf[b, r, pl.ds(0, 16)]
        for k in range(1, 8):
          p = p + stack[k] * ct_buf[b, r, pl.ds(16 * k, 16)]
        out_buf[b, r, pl.ds(0, 16)] = p
        for n in range(NEG):
          q = stack[0] * neg_buf[b, rl + n, pl.ds(0, 16)]
          for k in range(1, 8):
            q = q + stack[k] * neg_buf[b, rl + n, pl.ds(16 * k, 16)]
          out_buf[b, r, pl.ds(16 * (n + 1), 16)] = q
        return carry
      lax.fori_loop(0, C, row, 0)

    issue_chunk(0, 0)

    def body(i, carry):
      for b in (0, 1):
        c = 2 * i + b

        @pl.when(c + 1 < NCH)
        def _issue():
          issue_chunk(c + 1, 1 - b)

        wait_chunk(c, b)

        @pl.when(c >= 2)
        def _drain():
          out_copy(c - 2, b).wait()

        compute_chunk(b)
        out_copy(c, b).start()
      return carry

    lax.fori_loop(0, NCH // 2, body, 0)
    out_copy(NCH - 2, 0).wait()
    out_copy(NCH - 1, 1).wait()

  return k(target_emb, context_emb, output_emb, gidx, ctidx, ctxidx, negidx)


def _softplus(x):
  return jnp.maximum(x, 0.0) + jnp.log1p(jnp.exp(-jnp.abs(x)))


def _tc_loss(parts, gmat):
  nblk = 8
  rows = B // nblk

  def body(p_ref, g_ref, o_ref):
    i = pl.program_id(0)
    d = jnp.dot(p_ref[...], g_ref[...], preferred_element_type=jnp.float32)
    d = jnp.clip(d, -10.0, 10.0)
    part = (jnp.sum(_softplus(-d[:, 0:1])) +
            jnp.sum(_softplus(d[:, 1:1 + NEG])))

    @pl.when(i == 0)
    def _init():
      o_ref[0, 0] = 0.0

    o_ref[0, 0] += part

    @pl.when(i == nblk - 1)
    def _fin():
      o_ref[0, 0] = o_ref[0, 0] * (1.0 / B)

  out = pl.pallas_call(
      body,
      grid=(nblk,),
      in_specs=[
          pl.BlockSpec((rows, OUTW), lambda i: (i, 0)),
          pl.BlockSpec((OUTW, 1 + NEG), lambda i: (0, 0)),
      ],
      out_specs=pl.BlockSpec(memory_space=pltpu.SMEM),
      out_shape=jax.ShapeDtypeStruct((1, 1), jnp.float32),
  )(parts, gmat)
  return out[0, 0]


_GMAT = np.repeat(np.eye(1 + NEG, dtype=np.float32), 16, axis=0)


def kernel(target_emb, context_emb, output_emb, pos_graph_emb,
           pos_context_target, pos_contexts, pos_negatives):
  gidx = jnp.asarray(pos_graph_emb, jnp.int32)
  ctidx = jnp.asarray(pos_context_target, jnp.int32)
  ctxidx = jnp.asarray(pos_contexts, jnp.int32).reshape(-1)
  negidx = jnp.asarray(pos_negatives, jnp.int32).reshape(-1)
  parts = _sc_partials(target_emb, context_emb, output_emb,
                       gidx, ctidx, ctxidx, negidx)
  return _tc_loss(parts, jnp.asarray(_GMAT))
